# Initial kernel scaffold; baseline (speedup 1.0000x reference)
import jax
import jax.numpy as jnp
from jax.experimental import pallas as pl


def kernel(world_pos, image_feature, edge_index, is_training, read_intermediate, vis_att, W_enc1, b_enc1, W_enc2, b_enc2, W_eenc1, b_eenc1, W_eenc2, b_eenc2, A_s, A_r, A_e, W_msg, W_u1, b_u1, W_u2, b_u2, W_d1, b_d1, W_d2, b_d2):
    raise NotImplementedError("write your pallas kernel here")


# speedup vs baseline: 1.2261x; 1.2261x over previous
import functools

import jax
import jax.numpy as jnp
from jax import lax
from jax.experimental import pallas as pl
from jax.experimental.pallas import tpu as pltpu

N = 10000
D = 128
NBLK = 2000
EBLK = 16000


def _full(shape):
    return pl.BlockSpec(shape, lambda i: (0,) * len(shape))


def _encode_body(pos_ref, img_ref, W1p_ref, W1i_ref, b1_ref, W2_ref, b2_ref,
                 Wp_ref, h_ref, hw_ref, ab_ref):
    t = jnp.maximum(pos_ref[...] @ W1p_ref[...] + img_ref[...] @ W1i_ref[...]
                    + b1_ref[...], 0.0)
    h = t @ W2_ref[...] + b2_ref[...]
    h_ref[...] = h
    p = h @ Wp_ref[...]
    hw_ref[...] = p[:, :D]
    ab_ref[...] = p[:, D:]


def _encode(pos, img, W1, b1, W2, b2, Wp):
    grid = (N // NBLK,)
    return pl.pallas_call(
        _encode_body,
        grid=grid,
        in_specs=[
            pl.BlockSpec((NBLK, 3), lambda i: (i, 0)),
            pl.BlockSpec((NBLK, D), lambda i: (i, 0)),
            _full((3, D)), _full((D, D)), _full((1, D)),
            _full((D, D)), _full((1, D)), _full((D, D + 2)),
        ],
        out_specs=[
            pl.BlockSpec((NBLK, D), lambda i: (i, 0)),
            pl.BlockSpec((NBLK, D), lambda i: (i, 0)),
            pl.BlockSpec((NBLK, 2), lambda i: (i, 0)),
        ],
        out_shape=[
            jax.ShapeDtypeStruct((N, D), jnp.float32),
            jax.ShapeDtypeStruct((N, D), jnp.float32),
            jax.ShapeDtypeStruct((N, 2), jnp.float32),
        ],
    )(pos, img, W1[:3], W1[3:], b1[None], W2, b2[None], Wp)


def _ae_body(ef_ref, W1_ref, b1_ref, B_ref, c_ref, out_ref):
    t = jnp.maximum(ef_ref[...] @ W1_ref[...] + b1_ref[...], 0.0)
    out_ref[...] = t @ B_ref[...] + c_ref[...]


def _ae_table(efeat, W_eenc1, b_eenc1, B_e, c_e, E2, S):
    grid = (E2 // EBLK,)
    return pl.pallas_call(
        _ae_body,
        grid=grid,
        in_specs=[
            pl.BlockSpec((EBLK, 4), lambda i: (i, 0)),
            _full((4, D)), _full((1, D)), _full((D, S)), _full((1, S)),
        ],
        out_specs=pl.BlockSpec((EBLK, S), lambda i: (i, 0)),
        out_shape=jax.ShapeDtypeStruct((E2, S), jnp.float32),
    )(efeat, W_eenc1, b_eenc1[None], B_e, c_e)


def _update_body(h_ref, agg_ref, W1h_ref, W1a_ref, b1_ref, W2_ref, b2_ref,
                 Wp_ref, hn_ref, hw_ref, ab_ref):
    t = jnp.maximum(h_ref[...] @ W1h_ref[...] + agg_ref[...] @ W1a_ref[...]
                    + b1_ref[...], 0.0)
    hn = h_ref[...] + t @ W2_ref[...] + b2_ref[...]
    hn_ref[...] = hn
    p = hn @ Wp_ref[...]
    hw_ref[...] = p[:, :D]
    ab_ref[...] = p[:, D:]


def _update(h, agg, W1h, W1a, b1, W2, b2, Wp):
    grid = (N // NBLK,)
    return pl.pallas_call(
        _update_body,
        grid=grid,
        in_specs=[
            pl.BlockSpec((NBLK, D), lambda i: (i, 0)),
            pl.BlockSpec((NBLK, D), lambda i: (i, 0)),
            _full((D, D)), _full((D, D)), _full((1, D)),
            _full((D, D)), _full((1, D)), _full((D, D + 2)),
        ],
        out_specs=[
            pl.BlockSpec((NBLK, D), lambda i: (i, 0)),
            pl.BlockSpec((NBLK, D), lambda i: (i, 0)),
            pl.BlockSpec((NBLK, 2), lambda i: (i, 0)),
        ],
        out_shape=[
            jax.ShapeDtypeStruct((N, D), jnp.float32),
            jax.ShapeDtypeStruct((N, D), jnp.float32),
            jax.ShapeDtypeStruct((N, 2), jnp.float32),
        ],
    )(h, agg, W1h, W1a, b1[None], W2, b2[None], Wp)


def _decode_body(h_ref, W1_ref, b1_ref, W2_ref, b2_ref, out_ref):
    t = jnp.maximum(h_ref[...] @ W1_ref[...] + b1_ref[...], 0.0)
    out_ref[...] = t @ W2_ref[...] + b2_ref[...]


def _decode(h, W1, b1, W2, b2):
    grid = (N // NBLK,)
    return pl.pallas_call(
        _decode_body,
        grid=grid,
        in_specs=[
            pl.BlockSpec((NBLK, D), lambda i: (i, 0)),
            _full((D, D)), _full((1, D)), _full((D, 3)), _full((1, 3)),
        ],
        out_specs=pl.BlockSpec((NBLK, 3), lambda i: (i, 0)),
        out_shape=jax.ShapeDtypeStruct((N, 3), jnp.float32),
    )(h, W1, b1[None], W2, b2[None])


def kernel(world_pos, image_feature, edge_index, is_training,
           read_intermediate, vis_att,
           W_enc1, b_enc1, W_enc2, b_enc2, W_eenc1, b_eenc1, W_eenc2, b_eenc2,
           A_s, A_r, A_e, W_msg, W_u1, b_u1, W_u2, b_u2, W_d1, b_d1, W_d2, b_d2):
    S = A_s.shape[0]
    s0 = edge_index[0]
    r0 = edge_index[1]
    send = jnp.concatenate([s0, r0], 0)
    recv = jnp.concatenate([r0, s0], 0)
    E2 = send.shape[0]

    rel = world_pos[send] - world_pos[recv]
    nrm = jnp.linalg.norm(rel, axis=-1, keepdims=True)
    efeat = jnp.concatenate([rel, nrm], -1)

    B_e = W_eenc2 @ A_e.T
    c_e = (b_eenc2 @ A_e.T)[None]
    ae_all = _ae_table(efeat, W_eenc1, b_eenc1, B_e, c_e, E2, S)

    Wp = jnp.concatenate(
        [W_msg, A_s[:, :, None], A_r[:, :, None]], axis=2)

    h, hw, ab = _encode(world_pos, image_feature, W_enc1, b_enc1,
                        W_enc2, b_enc2, Wp[0])

    W1h = W_u1[:, :D, :]
    W1a = W_u1[:, D:, :]

    for i in range(S):
        a_s = ab[:, 0]
        a_r = ab[:, 1]
        logit = jax.nn.leaky_relu(a_s[send] + a_r[recv] + ae_all[:, i], 0.2)
        ex = jnp.exp(logit)
        den = jax.ops.segment_sum(ex, recv, num_segments=N)
        alpha = ex / (den[recv] + 1e-9)
        msg = alpha[:, None] * hw[send]
        agg = jax.ops.segment_sum(msg, recv, num_segments=N)
        Wp_next = Wp[i + 1] if i + 1 < S else Wp[0]
        h, hw, ab = _update(h, agg, W1h[i], W1a[i], b_u1[i], W_u2[i], b_u2[i],
                            Wp_next)

    return _decode(h, W_d1, b_d1, W_d2, b_d2)


# speedup vs baseline: 14.7637x; 12.0416x over previous
import functools

import jax
import jax.numpy as jnp
from jax import lax
from jax.experimental import pallas as pl
from jax.experimental.pallas import tpu as pltpu
from jax.experimental.pallas import tpu_sc as plsc

N = 10000
D = 128
NBLK = 2000
EBLK = 16000

NC = 2
NS = 16
NW = NC * NS
E2 = 320000
EW = E2 // NW
KC = 80
NCH = EW // KC
NROW = 640


def _full(shape):
    return pl.BlockSpec(shape, lambda i: (0,) * len(shape))


def _encode_body(pos_ref, img_ref, W1p_ref, W1i_ref, b1_ref, W2_ref, b2_ref,
                 Wp_ref, h_ref, hw_ref, ab_ref):
    t = jnp.maximum(pos_ref[...] @ W1p_ref[...] + img_ref[...] @ W1i_ref[...]
                    + b1_ref[...], 0.0)
    h = t @ W2_ref[...] + b2_ref[...]
    h_ref[...] = h
    p = h @ Wp_ref[...]
    hw_ref[...] = p[:, :D]
    ab_ref[...] = p[:, D:]


def _encode(pos, img, W1, b1, W2, b2, Wp):
    grid = (N // NBLK,)
    return pl.pallas_call(
        _encode_body,
        grid=grid,
        in_specs=[
            pl.BlockSpec((NBLK, 3), lambda i: (i, 0)),
            pl.BlockSpec((NBLK, D), lambda i: (i, 0)),
            _full((3, D)), _full((D, D)), _full((1, D)),
            _full((D, D)), _full((1, D)), _full((D, D + 2)),
        ],
        out_specs=[
            pl.BlockSpec((NBLK, D), lambda i: (i, 0)),
            pl.BlockSpec((NBLK, D), lambda i: (i, 0)),
            pl.BlockSpec((NBLK, 2), lambda i: (i, 0)),
        ],
        out_shape=[
            jax.ShapeDtypeStruct((N, D), jnp.float32),
            jax.ShapeDtypeStruct((N, D), jnp.float32),
            jax.ShapeDtypeStruct((N, 2), jnp.float32),
        ],
    )(pos, img, W1[:3], W1[3:], b1[None], W2, b2[None], Wp)


def _ae_body(ef_ref, W1_ref, b1_ref, B_ref, c_ref, out_ref):
    t = jnp.maximum(ef_ref[...] @ W1_ref[...] + b1_ref[...], 0.0)
    out_ref[...] = t @ B_ref[...] + c_ref[...]


def _ae_table(efeat, W_eenc1, b_eenc1, B_e, c_e, E2, S):
    grid = (E2 // EBLK,)
    return pl.pallas_call(
        _ae_body,
        grid=grid,
        in_specs=[
            pl.BlockSpec((EBLK, 4), lambda i: (i, 0)),
            _full((4, D)), _full((1, D)), _full((D, S)), _full((1, S)),
        ],
        out_specs=pl.BlockSpec((EBLK, S), lambda i: (i, 0)),
        out_shape=jax.ShapeDtypeStruct((E2, S), jnp.float32),
    )(efeat, W_eenc1, b_eenc1[None], B_e, c_e)


def _update_body(h_ref, a0_ref, a1_ref, d0_ref, d1_ref, W1h_ref, W1a_ref,
                 b1_ref, W2_ref, b2_ref, Wp_ref, hn_ref, hw_ref, ab_ref):
    den = d0_ref[...] + d1_ref[...] + 1e-9
    agg = (a0_ref[...] + a1_ref[...]) / den
    t = jnp.maximum(h_ref[...] @ W1h_ref[...] + agg @ W1a_ref[...]
                    + b1_ref[...], 0.0)
    hn = h_ref[...] + t @ W2_ref[...] + b2_ref[...]
    hn_ref[...] = hn
    p = hn @ Wp_ref[...]
    hw_ref[...] = p[:, :D]
    ab_ref[...] = p[:, D:]


def _update(h, a0, a1, d0, d1, W1h, W1a, b1, W2, b2, Wp):
    grid = (N // NBLK,)
    return pl.pallas_call(
        _update_body,
        grid=grid,
        in_specs=[
            pl.BlockSpec((NBLK, D), lambda i: (i, 0)),
            pl.BlockSpec((NBLK, D), lambda i: (i, 0)),
            pl.BlockSpec((NBLK, D), lambda i: (i, 0)),
            pl.BlockSpec((NBLK, 1), lambda i: (i, 0)),
            pl.BlockSpec((NBLK, 1), lambda i: (i, 0)),
            _full((D, D)), _full((D, D)), _full((1, D)),
            _full((D, D)), _full((1, D)), _full((D, D + 2)),
        ],
        out_specs=[
            pl.BlockSpec((NBLK, D), lambda i: (i, 0)),
            pl.BlockSpec((NBLK, D), lambda i: (i, 0)),
            pl.BlockSpec((NBLK, 2), lambda i: (i, 0)),
        ],
        out_shape=[
            jax.ShapeDtypeStruct((N, D), jnp.float32),
            jax.ShapeDtypeStruct((N, D), jnp.float32),
            jax.ShapeDtypeStruct((N, 2), jnp.float32),
        ],
    )(h, a0, a1, d0, d1, W1h, W1a, b1[None], W2, b2[None], Wp)


def _decode_body(h_ref, W1_ref, b1_ref, W2_ref, b2_ref, out_ref):
    t = jnp.maximum(h_ref[...] @ W1_ref[...] + b1_ref[...], 0.0)
    out_ref[...] = t @ W2_ref[...] + b2_ref[...]


def _decode(h, W1, b1, W2, b2):
    grid = (N // NBLK,)
    return pl.pallas_call(
        _decode_body,
        grid=grid,
        in_specs=[
            pl.BlockSpec((NBLK, D), lambda i: (i, 0)),
            _full((D, D)), _full((1, D)), _full((D, 3)), _full((1, 3)),
        ],
        out_specs=pl.BlockSpec((NBLK, 3), lambda i: (i, 0)),
        out_shape=jax.ShapeDtypeStruct((N, 3), jnp.float32),
    )(h, W1, b1[None], W2, b2[None])


_MESH = None


def _mesh():
    global _MESH
    if _MESH is None:
        _MESH = plsc.VectorSubcoreMesh(core_axis_name="c", subcore_axis_name="s")
    return _MESH


NP = NROW * 16


def _att_body(send_h, recv_h, ae_h, as_h, ar_h, ex_h, denp_h,
              send_v, recv_v, ae_v, ex_v, as_v, ar_v, denp_v, tmp_v, acc_v,
              parts_sh):
    c = lax.axis_index("c")
    s = lax.axis_index("s")
    wid = c * NS + s
    pltpu.sync_copy(send_h.at[wid], send_v)
    pltpu.sync_copy(recv_h.at[wid], recv_v)
    pltpu.sync_copy(ae_h.at[wid], ae_v)
    pltpu.sync_copy(as_h, as_v)
    pltpu.sync_copy(ar_h, ar_v)

    zero16 = jnp.zeros((16,), jnp.float32)

    def zbody(k, _):
        denp_v[pl.ds(k * 16, 16)] = zero16
        return 0
    lax.fori_loop(0, NROW, zbody, 0)

    def body(j, _):
        for t in range(KC // 16):
            sl = pl.ds(t * 16, 16)
            s16 = send_v[j, sl]
            r16 = recv_v[j, sl]
            x = (plsc.load_gather(as_v, [s16])
                 + plsc.load_gather(ar_v, [r16])
                 + ae_v[j, sl])
            l = jnp.where(x >= 0.0, x, 0.2 * x)
            e = jnp.exp(l)
            ex_v[pl.ds(j * KC + t * 16, 16)] = e
            plsc.addupdate_scatter(denp_v, [r16], e)
        return 0
    lax.fori_loop(0, NCH, body, 0)

    pltpu.sync_copy(ex_v, ex_h.at[pl.ds(wid * EW, EW)])
    pltpu.sync_copy(denp_v, parts_sh.at[s])
    plsc.subcore_barrier()
    seg = NP // NS

    def zacc(t, _):
        acc_v[pl.ds(t * 16, 16)] = zero16
        return 0
    lax.fori_loop(0, seg // 16, zacc, 0)
    for p in range(NS):
        pltpu.sync_copy(parts_sh.at[p, pl.ds(s * seg, seg)], tmp_v)

        def radd(t, _):
            sl = pl.ds(t * 16, 16)
            acc_v[sl] = acc_v[sl] + tmp_v[sl]
            return 0
        lax.fori_loop(0, seg // 16, radd, 0)
    pltpu.sync_copy(acc_v, denp_h.at[c, pl.ds(s * seg, seg)])


def _att_call(send_r, recv_r, ae_r, as_r, ar_r):
    return pl.kernel(
        _att_body,
        out_type=[
            jax.ShapeDtypeStruct((NW * EW,), jnp.float32),
            jax.ShapeDtypeStruct((NC, NP), jnp.float32),
        ],
        mesh=_mesh(),
        compiler_params=pltpu.CompilerParams(needs_layout_passes=False),
        scratch_types=[
            pltpu.VMEM((NCH, KC), jnp.int32),
            pltpu.VMEM((NCH, KC), jnp.int32),
            pltpu.VMEM((NCH, KC), jnp.float32),
            pltpu.VMEM((EW,), jnp.float32),
            pltpu.VMEM((NP,), jnp.float32),
            pltpu.VMEM((NP,), jnp.float32),
            pltpu.VMEM((NP,), jnp.float32),
            pltpu.VMEM((NP // NS,), jnp.float32),
            pltpu.VMEM((NP // NS,), jnp.float32),
            pltpu.VMEM_SHARED((NS, NP), jnp.float32),
        ],
    )(send_r, recv_r, ae_r, as_r, ar_r)


def _msg_body(send_h, recv_h, ex_h, hw_h, aggp_h,
              send_v, recv_v, ex_v, rows_v, sem, agg_sh):
    c = lax.axis_index("c")
    s = lax.axis_index("s")
    wid = c * NS + s
    pltpu.sync_copy(send_h.at[wid], send_v)
    pltpu.sync_copy(recv_h.at[wid], recv_v)

    zero16 = jnp.zeros((16,), jnp.float32)

    def zbody(k, _):
        rows_v[lax.shift_right_logical(k, 3),
               pl.ds(lax.bitwise_and(k, 7) * 16, 16)] = zero16
        return 0
    lax.fori_loop(0, KC * 8, zbody, 0)
    for q in range(8):
        pltpu.sync_copy(rows_v, agg_sh.at[pl.ds(s * 640 + q * 80, 80)])
    plsc.subcore_barrier()

    def mbody(j, _):
        pltpu.sync_copy(ex_h.at[pl.ds(wid * EW + j * KC, KC)], ex_v)
        pltpu.async_copy(hw_h.at[send_v.at[j]], rows_v, sem).wait()

        def rbody(k, _):
            a16 = plsc.load_gather(ex_v, [jnp.full((16,), k, jnp.int32)])
            for u in range(8):
                su = pl.ds(u * 16, 16)
                rows_v[k, su] = rows_v[k, su] * a16
            return 0
        lax.fori_loop(0, KC, rbody, 0)
        pltpu.sync_copy(rows_v, agg_sh.at[recv_v.at[j]], add=True)
        return 0
    lax.fori_loop(0, NCH, mbody, 0)
    plsc.subcore_barrier()

    for q in range(5):
        pltpu.sync_copy(agg_sh.at[pl.ds(s * 640 + q * 128, 128)],
                        aggp_h.at[c, pl.ds(s * 640 + q * 128, 128)])


def _msg_call(send_r, recv_r, ex_r, hw):
    return pl.kernel(
        _msg_body,
        out_type=jax.ShapeDtypeStruct((NC, NP, D), jnp.float32),
        mesh=_mesh(),
        compiler_params=pltpu.CompilerParams(needs_layout_passes=False),
        scratch_types=[
            pltpu.VMEM((NCH, KC), jnp.int32),
            pltpu.VMEM((NCH, KC), jnp.int32),
            pltpu.VMEM((KC,), jnp.float32),
            pltpu.VMEM((KC, D), jnp.float32),
            pltpu.SemaphoreType.DMA,
            pltpu.VMEM_SHARED((NP, D), jnp.float32),
        ],
    )(send_r, recv_r, ex_r, hw)


def kernel(world_pos, image_feature, edge_index, is_training,
           read_intermediate, vis_att,
           W_enc1, b_enc1, W_enc2, b_enc2, W_eenc1, b_eenc1, W_eenc2, b_eenc2,
           A_s, A_r, A_e, W_msg, W_u1, b_u1, W_u2, b_u2, W_d1, b_d1, W_d2, b_d2):
    S = A_s.shape[0]
    s0 = edge_index[0]
    r0 = edge_index[1]
    send = jnp.concatenate([s0, r0], 0)
    recv = jnp.concatenate([r0, s0], 0)
    E2 = send.shape[0]

    rel = world_pos[send] - world_pos[recv]
    nrm = jnp.linalg.norm(rel, axis=-1, keepdims=True)
    efeat = jnp.concatenate([rel, nrm], -1)

    B_e = W_eenc2 @ A_e.T
    c_e = (b_eenc2 @ A_e.T)[None]
    ae_all = _ae_table(efeat, W_eenc1, b_eenc1, B_e, c_e, E2, S)
    ae_T = ae_all.T

    send_r = send.astype(jnp.int32).reshape(NW, NCH, KC)
    recv_r = recv.astype(jnp.int32).reshape(NW, NCH, KC)

    Wp = jnp.concatenate(
        [W_msg, A_s[:, :, None], A_r[:, :, None]], axis=2)

    h, hw, ab = _encode(world_pos, image_feature, W_enc1, b_enc1,
                        W_enc2, b_enc2, Wp[0])

    W1h = W_u1[:, :D, :]
    W1a = W_u1[:, D:, :]

    for i in range(S):
        ae_r = ae_T[i].reshape(NW, NCH, KC)
        asr = jnp.zeros((2, NP), jnp.float32).at[:, :N].set(ab.T)
        ex_r, denp = _att_call(send_r, recv_r, ae_r, asr[0], asr[1])
        aggp = _msg_call(send_r, recv_r, ex_r, hw)
        Wp_next = Wp[i + 1] if i + 1 < S else Wp[0]
        h, hw, ab = _update(h, aggp[0, :N], aggp[1, :N],
                            denp[0, :N, None], denp[1, :N, None],
                            W1h[i], W1a[i], b_u1[i],
                            W_u2[i], b_u2[i], Wp_next)

    return _decode(h, W_d1, b_d1, W_d2, b_d2)


# speedup vs baseline: 20.7872x; 1.4080x over previous
import functools

import jax
import jax.numpy as jnp
from jax import lax
from jax.experimental import pallas as pl
from jax.experimental.pallas import tpu as pltpu
from jax.experimental.pallas import tpu_sc as plsc

N = 10000
D = 128
NBLK = 2000
EBLK = 16000

NC = 2
NS = 16
NW = NC * NS
E2 = 320000
EW = E2 // NW
KC = 80
NCH = EW // KC
KC2 = 40
NCH2 = EW // KC2
NROW = 640


def _full(shape):
    return pl.BlockSpec(shape, lambda i: (0,) * len(shape))


def _encode_body(pos_ref, img_ref, W1p_ref, W1i_ref, b1_ref, W2_ref, b2_ref,
                 Wp_ref, h_ref, hw_ref, ab_ref):
    t = jnp.maximum(pos_ref[...] @ W1p_ref[...] + img_ref[...] @ W1i_ref[...]
                    + b1_ref[...], 0.0)
    h = t @ W2_ref[...] + b2_ref[...]
    h_ref[...] = h
    p = h @ Wp_ref[...]
    hw_ref[...] = p[:, :D]
    ab_ref[...] = p[:, D:]


def _encode(pos, img, W1, b1, W2, b2, Wp):
    grid = (N // NBLK,)
    return pl.pallas_call(
        _encode_body,
        grid=grid,
        in_specs=[
            pl.BlockSpec((NBLK, 3), lambda i: (i, 0)),
            pl.BlockSpec((NBLK, D), lambda i: (i, 0)),
            _full((3, D)), _full((D, D)), _full((1, D)),
            _full((D, D)), _full((1, D)), _full((D, D + 2)),
        ],
        out_specs=[
            pl.BlockSpec((NBLK, D), lambda i: (i, 0)),
            pl.BlockSpec((NBLK, D), lambda i: (i, 0)),
            pl.BlockSpec((NBLK, 2), lambda i: (i, 0)),
        ],
        out_shape=[
            jax.ShapeDtypeStruct((N, D), jnp.float32),
            jax.ShapeDtypeStruct((N, D), jnp.float32),
            jax.ShapeDtypeStruct((N, 2), jnp.float32),
        ],
    )(pos, img, W1[:3], W1[3:], b1[None], W2, b2[None], Wp)


def _ae_body(ef_ref, W1_ref, b1_ref, B_ref, c_ref, out_ref):
    t = jnp.maximum(ef_ref[...] @ W1_ref[...] + b1_ref[...], 0.0)
    out_ref[...] = t @ B_ref[...] + c_ref[...]


def _ae_table(efeat, W_eenc1, b_eenc1, B_e, c_e, E2, S):
    grid = (E2 // EBLK,)
    return pl.pallas_call(
        _ae_body,
        grid=grid,
        in_specs=[
            pl.BlockSpec((EBLK, 4), lambda i: (i, 0)),
            _full((4, D)), _full((1, D)), _full((D, S)), _full((1, S)),
        ],
        out_specs=pl.BlockSpec((EBLK, S), lambda i: (i, 0)),
        out_shape=jax.ShapeDtypeStruct((E2, S), jnp.float32),
    )(efeat, W_eenc1, b_eenc1[None], B_e, c_e)


def _update_body(h_ref, a0_ref, a1_ref, d0_ref, d1_ref, W1h_ref, W1a_ref,
                 b1_ref, W2_ref, b2_ref, Wp_ref, hn_ref, hw_ref, ab_ref):
    den = d0_ref[...] + d1_ref[...] + 1e-9
    agg = (a0_ref[0] + a1_ref[0]) / den
    t = jnp.maximum(h_ref[...] @ W1h_ref[...] + agg @ W1a_ref[...]
                    + b1_ref[...], 0.0)
    hn = h_ref[...] + t @ W2_ref[...] + b2_ref[...]
    hn_ref[...] = hn
    p = hn @ Wp_ref[...]
    hw_ref[...] = p[:, :D]
    ab_ref[...] = p[:, D:]


def _update(h, a0, a1, d0, d1, W1h, W1a, b1, W2, b2, Wp):
    grid = (N // NBLK,)
    return pl.pallas_call(
        _update_body,
        grid=grid,
        in_specs=[
            pl.BlockSpec((NBLK, D), lambda i: (i, 0)),
            pl.BlockSpec((1, NBLK, D), lambda i: (0, i, 0)),
            pl.BlockSpec((1, NBLK, D), lambda i: (1, i, 0)),
            pl.BlockSpec((NBLK, 1), lambda i: (i, 0)),
            pl.BlockSpec((NBLK, 1), lambda i: (i, 0)),
            _full((D, D)), _full((D, D)), _full((1, D)),
            _full((D, D)), _full((1, D)), _full((D, D + 2)),
        ],
        out_specs=[
            pl.BlockSpec((NBLK, D), lambda i: (i, 0)),
            pl.BlockSpec((NBLK, D), lambda i: (i, 0)),
            pl.BlockSpec((NBLK, 2), lambda i: (i, 0)),
        ],
        out_shape=[
            jax.ShapeDtypeStruct((N, D), jnp.float32),
            jax.ShapeDtypeStruct((N, D), jnp.float32),
            jax.ShapeDtypeStruct((N, 2), jnp.float32),
        ],
    )(h, a0, a1, d0, d1, W1h, W1a, b1[None], W2, b2[None], Wp)


def _decode_body(h_ref, W1_ref, b1_ref, W2_ref, b2_ref, out_ref):
    t = jnp.maximum(h_ref[...] @ W1_ref[...] + b1_ref[...], 0.0)
    out_ref[...] = t @ W2_ref[...] + b2_ref[...]


def _decode(h, W1, b1, W2, b2):
    grid = (N // NBLK,)
    return pl.pallas_call(
        _decode_body,
        grid=grid,
        in_specs=[
            pl.BlockSpec((NBLK, D), lambda i: (i, 0)),
            _full((D, D)), _full((1, D)), _full((D, 3)), _full((1, 3)),
        ],
        out_specs=pl.BlockSpec((NBLK, 3), lambda i: (i, 0)),
        out_shape=jax.ShapeDtypeStruct((N, 3), jnp.float32),
    )(h, W1, b1[None], W2, b2[None])


_MESH = None


def _mesh():
    global _MESH
    if _MESH is None:
        _MESH = plsc.VectorSubcoreMesh(core_axis_name="c", subcore_axis_name="s")
    return _MESH


NP = NROW * 16


def _att_body(send_h, recv_h, ae_h, as_h, ar_h, ex_h, denp_h,
              send_v, recv_v, ae_v, ex_v, as_v, ar_v, denp_v, tmp_v, acc_v,
              parts_sh):
    c = lax.axis_index("c")
    s = lax.axis_index("s")
    wid = c * NS + s
    pltpu.sync_copy(send_h.at[wid], send_v)
    pltpu.sync_copy(recv_h.at[wid], recv_v)
    pltpu.sync_copy(ae_h.at[wid], ae_v)
    pltpu.sync_copy(as_h, as_v)
    pltpu.sync_copy(ar_h, ar_v)

    zero16 = jnp.zeros((16,), jnp.float32)

    def zbody(k, _):
        denp_v[pl.ds(k * 16, 16)] = zero16
        return 0
    lax.fori_loop(0, NROW, zbody, 0)

    def body(j, _):
        for t in range(KC // 16):
            sl = pl.ds(t * 16, 16)
            s16 = send_v[j, sl]
            r16 = recv_v[j, sl]
            x = (plsc.load_gather(as_v, [s16])
                 + plsc.load_gather(ar_v, [r16])
                 + ae_v[j, sl])
            l = jnp.where(x >= 0.0, x, 0.2 * x)
            e = jnp.exp(l)
            ex_v[pl.ds(j * KC + t * 16, 16)] = e
            plsc.addupdate_scatter(denp_v, [r16], e)
        return 0
    lax.fori_loop(0, NCH, body, 0)

    pltpu.sync_copy(ex_v, ex_h.at[pl.ds(wid * EW, EW)])
    pltpu.sync_copy(denp_v, parts_sh.at[s])
    plsc.subcore_barrier()
    seg = NP // NS

    def zacc(t, _):
        acc_v[pl.ds(t * 16, 16)] = zero16
        return 0
    lax.fori_loop(0, seg // 16, zacc, 0)
    for p in range(NS):
        pltpu.sync_copy(parts_sh.at[p, pl.ds(s * seg, seg)], tmp_v)

        def radd(t, _):
            sl = pl.ds(t * 16, 16)
            acc_v[sl] = acc_v[sl] + tmp_v[sl]
            return 0
        lax.fori_loop(0, seg // 16, radd, 0)
    pltpu.sync_copy(acc_v, denp_h.at[c, pl.ds(s * seg, seg)])


_ATT_K = None


def _att_call(send_r, recv_r, ae_r, as_r, ar_r):
    global _ATT_K
    if _ATT_K is None:
        _ATT_K = _make_att()
    return _ATT_K(send_r, recv_r, ae_r, as_r, ar_r)


def _make_att():
    return pl.kernel(
        _att_body,
        out_type=[
            jax.ShapeDtypeStruct((NW * EW,), jnp.float32),
            jax.ShapeDtypeStruct((NC, NP), jnp.float32),
        ],
        mesh=_mesh(),
        compiler_params=pltpu.CompilerParams(needs_layout_passes=False),
        scratch_types=[
            pltpu.VMEM((NCH, KC), jnp.int32),
            pltpu.VMEM((NCH, KC), jnp.int32),
            pltpu.VMEM((NCH, KC), jnp.float32),
            pltpu.VMEM((EW,), jnp.float32),
            pltpu.VMEM((NP,), jnp.float32),
            pltpu.VMEM((NP,), jnp.float32),
            pltpu.VMEM((NP,), jnp.float32),
            pltpu.VMEM((NP // NS,), jnp.float32),
            pltpu.VMEM((NP // NS,), jnp.float32),
            pltpu.VMEM_SHARED((NS, NP), jnp.float32),
        ],
    )


def _msg_body(send_h, recv_h, ex_h, hw_h, aggp_h,
              recv_v, sb0, sb1, sb2, eb0, eb1, eb2, r0_v, r1_v, r2_v,
              i0, i1, i2, g0, g1, g2, s0, s1, s2, agg_sh):
    c = lax.axis_index("c")
    s = lax.axis_index("s")
    wid = c * NS + s
    pltpu.sync_copy(recv_h.at[wid], recv_v)

    sendb = (sb0, sb1, sb2)
    exb = (eb0, eb1, eb2)
    rows = (r0_v, r1_v, r2_v)
    isem = (i0, i1, i2)
    gsem = (g0, g1, g2)
    ssem = (s0, s1, s2)

    zero16 = jnp.zeros((16,), jnp.float32)

    def zbody(k, _):
        r0_v[lax.shift_right_logical(k, 3),
             pl.ds(lax.bitwise_and(k, 7) * 16, 16)] = zero16
        return 0
    lax.fori_loop(0, KC2 * 8, zbody, 0)
    for q in range(640 // KC2):
        pltpu.sync_copy(r0_v, agg_sh.at[pl.ds(s * 640 + q * KC2, KC2)])
    plsc.subcore_barrier()

    def fire_ise(j, b):
        pltpu.async_copy(send_h.at[wid, j], sendb[b], isem[b])
        pltpu.async_copy(ex_h.at[pl.ds(wid * EW + j * KC2, KC2)], exb[b],
                         isem[b])

    def wait_ise(b):
        pltpu.make_async_copy(send_h.at[wid, 0], sendb[b], isem[b]).wait()
        pltpu.make_async_copy(ex_h.at[pl.ds(0, KC2)], exb[b], isem[b]).wait()

    def fire_g(b):
        pltpu.async_copy(hw_h.at[sendb[b]], rows[b], gsem[b])

    def wait_g(b):
        pltpu.make_async_copy(hw_h.at[sendb[b]], rows[b], gsem[b]).wait()

    def fire_s(j, b):
        pltpu.async_copy(rows[b], agg_sh.at[recv_v.at[j]], ssem[b], add=True)

    def wait_s(b):
        pltpu.make_async_copy(rows[b], agg_sh.at[recv_v.at[0]],
                              ssem[b]).wait()

    fire_ise(0, 0)
    fire_ise(1, 1)
    wait_ise(0)
    fire_g(0)

    def qbody(q, _):
        for b in range(3):
            j = 3 * q + b
            b1 = (b + 1) % 3
            b2 = (b + 2) % 3

            @pl.when(j + 1 < NCH2)
            def _():
                wait_ise(b1)

                @pl.when(j >= 2)
                def _():
                    wait_s(b1)
                fire_g(b1)

            @pl.when(j + 2 < NCH2)
            def _():
                fire_ise(j + 2, b2)

            @pl.when(j < NCH2)
            def _():
                wait_g(b)

                def rbody(k, _):
                    a16 = plsc.load_gather(exb[b],
                                           [jnp.full((16,), k, jnp.int32)])
                    for u in range(8):
                        su = pl.ds(u * 16, 16)
                        rows[b][k, su] = rows[b][k, su] * a16
                    return 0
                lax.fori_loop(0, KC2, rbody, 0)
                fire_s(j, b)
        return 0
    lax.fori_loop(0, (NCH2 + 2) // 3, qbody, 0)
    wait_s((NCH2 - 3) % 3)
    wait_s((NCH2 - 2) % 3)
    wait_s((NCH2 - 1) % 3)
    plsc.subcore_barrier()

    for q in range(5):
        pltpu.sync_copy(agg_sh.at[pl.ds(s * 640 + q * 128, 128)],
                        aggp_h.at[c, pl.ds(s * 640 + q * 128, 128)])


_MSG_K = None


def _msg_call(send_r, recv_r, ex_r, hw):
    global _MSG_K
    if _MSG_K is None:
        _MSG_K = _make_msg()
    return _MSG_K(send_r, recv_r, ex_r, hw)


def _make_msg():
    return pl.kernel(
        _msg_body,
        out_type=jax.ShapeDtypeStruct((NC, NP, D), jnp.float32),
        mesh=_mesh(),
        compiler_params=pltpu.CompilerParams(needs_layout_passes=False),
        scratch_types=[
            pltpu.VMEM((NCH2, KC2), jnp.int32),
            pltpu.VMEM((KC2,), jnp.int32),
            pltpu.VMEM((KC2,), jnp.int32),
            pltpu.VMEM((KC2,), jnp.int32),
            pltpu.VMEM((KC2,), jnp.float32),
            pltpu.VMEM((KC2,), jnp.float32),
            pltpu.VMEM((KC2,), jnp.float32),
            pltpu.VMEM((KC2, D), jnp.float32),
            pltpu.VMEM((KC2, D), jnp.float32),
            pltpu.VMEM((KC2, D), jnp.float32),
            pltpu.SemaphoreType.DMA,
            pltpu.SemaphoreType.DMA,
            pltpu.SemaphoreType.DMA,
            pltpu.SemaphoreType.DMA,
            pltpu.SemaphoreType.DMA,
            pltpu.SemaphoreType.DMA,
            pltpu.SemaphoreType.DMA,
            pltpu.SemaphoreType.DMA,
            pltpu.SemaphoreType.DMA,
            pltpu.VMEM_SHARED((NP, D), jnp.float32),
        ],
    )


def kernel(world_pos, image_feature, edge_index, is_training,
           read_intermediate, vis_att,
           W_enc1, b_enc1, W_enc2, b_enc2, W_eenc1, b_eenc1, W_eenc2, b_eenc2,
           A_s, A_r, A_e, W_msg, W_u1, b_u1, W_u2, b_u2, W_d1, b_d1, W_d2, b_d2):
    S = A_s.shape[0]
    s0 = edge_index[0]
    r0 = edge_index[1]
    send = jnp.concatenate([s0, r0], 0)
    recv = jnp.concatenate([r0, s0], 0)
    E2 = send.shape[0]

    rel = world_pos[send] - world_pos[recv]
    nrm = jnp.linalg.norm(rel, axis=-1, keepdims=True)
    efeat = jnp.concatenate([rel, nrm], -1)

    B_e = W_eenc2 @ A_e.T
    c_e = (b_eenc2 @ A_e.T)[None]
    ae_all = _ae_table(efeat, W_eenc1, b_eenc1, B_e, c_e, E2, S)
    ae_T = ae_all.T

    send_r = send.astype(jnp.int32).reshape(NW, NCH, KC)
    recv_r = recv.astype(jnp.int32).reshape(NW, NCH, KC)
    send_r2 = send.astype(jnp.int32).reshape(NW, NCH2, KC2)
    recv_r2 = recv.astype(jnp.int32).reshape(NW, NCH2, KC2)

    Wp = jnp.concatenate(
        [W_msg, A_s[:, :, None], A_r[:, :, None]], axis=2)

    h, hw, ab = _encode(world_pos, image_feature, W_enc1, b_enc1,
                        W_enc2, b_enc2, Wp[0])

    W1h = W_u1[:, :D, :]
    W1a = W_u1[:, D:, :]

    for i in range(S):
        ae_r = ae_T[i].reshape(NW, NCH, KC)
        asr = jnp.zeros((2, NP), jnp.float32).at[:, :N].set(ab.T)
        ex_r, denp = _att_call(send_r, recv_r, ae_r, asr[0], asr[1])
        aggp = _msg_call(send_r2, recv_r2, ex_r, hw)
        Wp_next = Wp[i + 1] if i + 1 < S else Wp[0]
        h, hw, ab = _update(h, aggp, aggp,
                            denp[0, :N, None], denp[1, :N, None],
                            W1h[i], W1a[i], b_u1[i],
                            W_u2[i], b_u2[i], Wp_next)

    return _decode(h, W_d1, b_d1, W_d2, b_d2)


# speedup vs baseline: 20.9272x; 1.0067x over previous
import functools

import jax
import jax.numpy as jnp
from jax import lax
from jax.experimental import pallas as pl
from jax.experimental.pallas import tpu as pltpu
from jax.experimental.pallas import tpu_sc as plsc

N = 10000
D = 128
NBLK = 2000
EBLK = 16000

NC = 2
NS = 16
NW = NC * NS
E2 = 320000
EW = E2 // NW
KC = 80
NCH = EW // KC
KC2 = 40
NCH2 = EW // KC2
NROW = 640


def _full(shape):
    return pl.BlockSpec(shape, lambda i: (0,) * len(shape))


def _encode_body(pos_ref, img_ref, W1p_ref, W1i_ref, b1_ref, W2_ref, b2_ref,
                 Wp_ref, h_ref, hw_ref, ab_ref):
    t = jnp.maximum(pos_ref[...] @ W1p_ref[...] + img_ref[...] @ W1i_ref[...]
                    + b1_ref[...], 0.0)
    h = t @ W2_ref[...] + b2_ref[...]
    h_ref[...] = h
    p = h @ Wp_ref[...]
    hw_ref[...] = p[:, :D]
    ab_ref[...] = p[:, D:]


def _encode(pos, img, W1, b1, W2, b2, Wp):
    grid = (N // NBLK,)
    return pl.pallas_call(
        _encode_body,
        grid=grid,
        in_specs=[
            pl.BlockSpec((NBLK, 3), lambda i: (i, 0)),
            pl.BlockSpec((NBLK, D), lambda i: (i, 0)),
            _full((3, D)), _full((D, D)), _full((1, D)),
            _full((D, D)), _full((1, D)), _full((D, D + 2)),
        ],
        out_specs=[
            pl.BlockSpec((NBLK, D), lambda i: (i, 0)),
            pl.BlockSpec((NBLK, D), lambda i: (i, 0)),
            pl.BlockSpec((NBLK, 2), lambda i: (i, 0)),
        ],
        out_shape=[
            jax.ShapeDtypeStruct((N, D), jnp.float32),
            jax.ShapeDtypeStruct((N, D), jnp.float32),
            jax.ShapeDtypeStruct((N, 2), jnp.float32),
        ],
    )(pos, img, W1[:3], W1[3:], b1[None], W2, b2[None], Wp)


def _ae_body(ef_ref, W1_ref, b1_ref, B_ref, c_ref, out_ref):
    t = jnp.maximum(ef_ref[...] @ W1_ref[...] + b1_ref[...], 0.0)
    out_ref[...] = t @ B_ref[...] + c_ref[...]


def _ae_table(efeat, W_eenc1, b_eenc1, B_e, c_e, E2, S):
    grid = (E2 // EBLK,)
    return pl.pallas_call(
        _ae_body,
        grid=grid,
        in_specs=[
            pl.BlockSpec((EBLK, 4), lambda i: (i, 0)),
            _full((4, D)), _full((1, D)), _full((D, S)), _full((1, S)),
        ],
        out_specs=pl.BlockSpec((EBLK, S), lambda i: (i, 0)),
        out_shape=jax.ShapeDtypeStruct((E2, S), jnp.float32),
    )(efeat, W_eenc1, b_eenc1[None], B_e, c_e)


def _update_body(h_ref, a0_ref, a1_ref, d0_ref, d1_ref, W1h_ref, W1a_ref,
                 b1_ref, W2_ref, b2_ref, Wp_ref, hn_ref, hw_ref, ab_ref):
    den = d0_ref[...] + d1_ref[...] + 1e-9
    agg = (a0_ref[0] + a1_ref[0]) / den
    t = jnp.maximum(h_ref[...] @ W1h_ref[...] + agg @ W1a_ref[...]
                    + b1_ref[...], 0.0)
    hn = h_ref[...] + t @ W2_ref[...] + b2_ref[...]
    hn_ref[...] = hn
    p = hn @ Wp_ref[...]
    hw_ref[...] = p[:, :D]
    ab_ref[...] = p[:, D:]


def _update(h, a0, a1, d0, d1, W1h, W1a, b1, W2, b2, Wp):
    grid = (N // NBLK,)
    return pl.pallas_call(
        _update_body,
        grid=grid,
        in_specs=[
            pl.BlockSpec((NBLK, D), lambda i: (i, 0)),
            pl.BlockSpec((1, NBLK, D), lambda i: (0, i, 0)),
            pl.BlockSpec((1, NBLK, D), lambda i: (1, i, 0)),
            pl.BlockSpec((NBLK, 1), lambda i: (i, 0)),
            pl.BlockSpec((NBLK, 1), lambda i: (i, 0)),
            _full((D, D)), _full((D, D)), _full((1, D)),
            _full((D, D)), _full((1, D)), _full((D, D + 2)),
        ],
        out_specs=[
            pl.BlockSpec((NBLK, D), lambda i: (i, 0)),
            pl.BlockSpec((NBLK, D), lambda i: (i, 0)),
            pl.BlockSpec((NBLK, 2), lambda i: (i, 0)),
        ],
        out_shape=[
            jax.ShapeDtypeStruct((N, D), jnp.float32),
            jax.ShapeDtypeStruct((N, D), jnp.float32),
            jax.ShapeDtypeStruct((N, 2), jnp.float32),
        ],
    )(h, a0, a1, d0, d1, W1h, W1a, b1[None], W2, b2[None], Wp)


def _decode_body(h_ref, W1_ref, b1_ref, W2_ref, b2_ref, out_ref):
    t = jnp.maximum(h_ref[...] @ W1_ref[...] + b1_ref[...], 0.0)
    out_ref[...] = t @ W2_ref[...] + b2_ref[...]


def _decode(h, W1, b1, W2, b2):
    grid = (N // NBLK,)
    return pl.pallas_call(
        _decode_body,
        grid=grid,
        in_specs=[
            pl.BlockSpec((NBLK, D), lambda i: (i, 0)),
            _full((D, D)), _full((1, D)), _full((D, 3)), _full((1, 3)),
        ],
        out_specs=pl.BlockSpec((NBLK, 3), lambda i: (i, 0)),
        out_shape=jax.ShapeDtypeStruct((N, 3), jnp.float32),
    )(h, W1, b1[None], W2, b2[None])


_MESH = None


def _mesh():
    global _MESH
    if _MESH is None:
        _MESH = plsc.VectorSubcoreMesh(core_axis_name="c", subcore_axis_name="s")
    return _MESH


NP = NROW * 16


def _att_body(send_h, recv_h, ae_h, ab_h, ex_h, denp_h,
              send_v, recv_v, ae_v, ex_v, ab_v, denp_v, tmp_v, acc_v,
              parts_sh):
    c = lax.axis_index("c")
    s = lax.axis_index("s")
    wid = c * NS + s
    pltpu.sync_copy(send_h.at[wid], send_v)
    pltpu.sync_copy(recv_h.at[wid], recv_v)
    pltpu.sync_copy(ae_h.at[wid], ae_v)
    pltpu.sync_copy(ab_h, ab_v)

    zero16 = jnp.zeros((16,), jnp.float32)

    def zbody(k, _):
        denp_v[pl.ds(k * 16, 16)] = zero16
        return 0
    lax.fori_loop(0, NROW, zbody, 0)

    def body(j, _):
        for t in range(KC // 16):
            sl = pl.ds(t * 16, 16)
            s16 = send_v[j, sl]
            r16 = recv_v[j, sl]
            x = (plsc.load_gather(ab_v, [s16 * 2])
                 + plsc.load_gather(ab_v, [r16 * 2 + 1])
                 + ae_v[j, sl])
            l = jnp.where(x >= 0.0, x, 0.2 * x)
            e = jnp.exp(l)
            ex_v[pl.ds(j * KC + t * 16, 16)] = e
            plsc.addupdate_scatter(denp_v, [r16], e)
        return 0
    lax.fori_loop(0, NCH, body, 0)

    pltpu.sync_copy(ex_v, ex_h.at[pl.ds(wid * EW, EW)])
    pltpu.sync_copy(denp_v, parts_sh.at[s])
    plsc.subcore_barrier()
    seg = NP // NS

    def zacc(t, _):
        acc_v[pl.ds(t * 16, 16)] = zero16
        return 0
    lax.fori_loop(0, seg // 16, zacc, 0)
    for p in range(NS):
        pltpu.sync_copy(parts_sh.at[p, pl.ds(s * seg, seg)], tmp_v)

        def radd(t, _):
            sl = pl.ds(t * 16, 16)
            acc_v[sl] = acc_v[sl] + tmp_v[sl]
            return 0
        lax.fori_loop(0, seg // 16, radd, 0)
    pltpu.sync_copy(acc_v, denp_h.at[c, pl.ds(s * seg, seg)])


_ATT_K = None


def _att_call(send_r, recv_r, ae_r, ab_flat):
    global _ATT_K
    if _ATT_K is None:
        _ATT_K = _make_att()
    return _ATT_K(send_r, recv_r, ae_r, ab_flat)


def _make_att():
    return pl.kernel(
        _att_body,
        out_type=[
            jax.ShapeDtypeStruct((NW * EW,), jnp.float32),
            jax.ShapeDtypeStruct((NC, NP), jnp.float32),
        ],
        mesh=_mesh(),
        compiler_params=pltpu.CompilerParams(needs_layout_passes=False),
        scratch_types=[
            pltpu.VMEM((NCH, KC), jnp.int32),
            pltpu.VMEM((NCH, KC), jnp.int32),
            pltpu.VMEM((NCH, KC), jnp.float32),
            pltpu.VMEM((EW,), jnp.float32),
            pltpu.VMEM((2 * N,), jnp.float32),
            pltpu.VMEM((NP,), jnp.float32),
            pltpu.VMEM((NP // NS,), jnp.float32),
            pltpu.VMEM((NP // NS,), jnp.float32),
            pltpu.VMEM_SHARED((NS, NP), jnp.float32),
        ],
    )


def _msg_body(send_h, recv_h, ex_h, hw_h, aggp_h,
              recv_v, sb0, sb1, sb2, eb0, eb1, eb2, r0_v, r1_v, r2_v,
              i0, i1, i2, g0, g1, g2, s0, s1, s2, agg_sh):
    c = lax.axis_index("c")
    s = lax.axis_index("s")
    wid = c * NS + s
    pltpu.sync_copy(recv_h.at[wid], recv_v)

    sendb = (sb0, sb1, sb2)
    exb = (eb0, eb1, eb2)
    rows = (r0_v, r1_v, r2_v)
    isem = (i0, i1, i2)
    gsem = (g0, g1, g2)
    ssem = (s0, s1, s2)

    zero16 = jnp.zeros((16,), jnp.float32)

    def zbody(k, _):
        r0_v[lax.shift_right_logical(k, 3),
             pl.ds(lax.bitwise_and(k, 7) * 16, 16)] = zero16
        return 0
    lax.fori_loop(0, KC2 * 8, zbody, 0)
    for q in range(640 // KC2):
        pltpu.sync_copy(r0_v, agg_sh.at[pl.ds(s * 640 + q * KC2, KC2)])
    plsc.subcore_barrier()

    def fire_ise(j, b):
        pltpu.async_copy(send_h.at[wid, j], sendb[b], isem[b])
        pltpu.async_copy(ex_h.at[pl.ds(wid * EW + j * KC2, KC2)], exb[b],
                         isem[b])

    def wait_ise(b):
        pltpu.make_async_copy(send_h.at[wid, 0], sendb[b], isem[b]).wait()
        pltpu.make_async_copy(ex_h.at[pl.ds(0, KC2)], exb[b], isem[b]).wait()

    def fire_g(b):
        pltpu.async_copy(hw_h.at[sendb[b]], rows[b], gsem[b])

    def wait_g(b):
        pltpu.make_async_copy(hw_h.at[sendb[b]], rows[b], gsem[b]).wait()

    def fire_s(j, b):
        pltpu.async_copy(rows[b], agg_sh.at[recv_v.at[j]], ssem[b], add=True)

    def wait_s(b):
        pltpu.make_async_copy(rows[b], agg_sh.at[recv_v.at[0]],
                              ssem[b]).wait()

    fire_ise(0, 0)
    fire_ise(1, 1)
    wait_ise(0)
    fire_g(0)

    def qbody(q, _):
        for b in range(3):
            j = 3 * q + b
            b1 = (b + 1) % 3
            b2 = (b + 2) % 3

            @pl.when(j + 1 < NCH2)
            def _():
                wait_ise(b1)

                @pl.when(j >= 2)
                def _():
                    wait_s(b1)
                fire_g(b1)

            @pl.when(j + 2 < NCH2)
            def _():
                fire_ise(j + 2, b2)

            @pl.when(j < NCH2)
            def _():
                wait_g(b)

                def rbody(k, _):
                    a16 = plsc.load_gather(exb[b],
                                           [jnp.full((16,), k, jnp.int32)])
                    for u in range(8):
                        su = pl.ds(u * 16, 16)
                        rows[b][k, su] = rows[b][k, su] * a16
                    return 0
                lax.fori_loop(0, KC2, rbody, 0)
                fire_s(j, b)
        return 0
    lax.fori_loop(0, (NCH2 + 2) // 3, qbody, 0)
    wait_s((NCH2 - 3) % 3)
    wait_s((NCH2 - 2) % 3)
    wait_s((NCH2 - 1) % 3)
    plsc.subcore_barrier()

    for q in range(5):
        pltpu.sync_copy(agg_sh.at[pl.ds(s * 640 + q * 128, 128)],
                        aggp_h.at[c, pl.ds(s * 640 + q * 128, 128)])


_MSG_K = None


def _msg_call(send_r, recv_r, ex_r, hw):
    global _MSG_K
    if _MSG_K is None:
        _MSG_K = _make_msg()
    return _MSG_K(send_r, recv_r, ex_r, hw)


def _make_msg():
    return pl.kernel(
        _msg_body,
        out_type=jax.ShapeDtypeStruct((NC, NP, D), jnp.float32),
        mesh=_mesh(),
        compiler_params=pltpu.CompilerParams(needs_layout_passes=False),
        scratch_types=[
            pltpu.VMEM((NCH2, KC2), jnp.int32),
            pltpu.VMEM((KC2,), jnp.int32),
            pltpu.VMEM((KC2,), jnp.int32),
            pltpu.VMEM((KC2,), jnp.int32),
            pltpu.VMEM((KC2,), jnp.float32),
            pltpu.VMEM((KC2,), jnp.float32),
            pltpu.VMEM((KC2,), jnp.float32),
            pltpu.VMEM((KC2, D), jnp.float32),
            pltpu.VMEM((KC2, D), jnp.float32),
            pltpu.VMEM((KC2, D), jnp.float32),
            pltpu.SemaphoreType.DMA,
            pltpu.SemaphoreType.DMA,
            pltpu.SemaphoreType.DMA,
            pltpu.SemaphoreType.DMA,
            pltpu.SemaphoreType.DMA,
            pltpu.SemaphoreType.DMA,
            pltpu.SemaphoreType.DMA,
            pltpu.SemaphoreType.DMA,
            pltpu.SemaphoreType.DMA,
            pltpu.VMEM_SHARED((NP, D), jnp.float32),
        ],
    )


def kernel(world_pos, image_feature, edge_index, is_training,
           read_intermediate, vis_att,
           W_enc1, b_enc1, W_enc2, b_enc2, W_eenc1, b_eenc1, W_eenc2, b_eenc2,
           A_s, A_r, A_e, W_msg, W_u1, b_u1, W_u2, b_u2, W_d1, b_d1, W_d2, b_d2):
    S = A_s.shape[0]
    s0 = edge_index[0]
    r0 = edge_index[1]
    send = jnp.concatenate([s0, r0], 0)
    recv = jnp.concatenate([r0, s0], 0)
    E2 = send.shape[0]

    rel = world_pos[send] - world_pos[recv]
    nrm = jnp.linalg.norm(rel, axis=-1, keepdims=True)
    efeat = jnp.concatenate([rel, nrm], -1)

    B_e = W_eenc2 @ A_e.T
    c_e = (b_eenc2 @ A_e.T)[None]
    ae_all = _ae_table(efeat, W_eenc1, b_eenc1, B_e, c_e, E2, S)
    ae_T = ae_all.T

    send_r = send.astype(jnp.int32).reshape(NW, NCH, KC)
    recv_r = recv.astype(jnp.int32).reshape(NW, NCH, KC)
    send_r2 = send.astype(jnp.int32).reshape(NW, NCH2, KC2)
    recv_r2 = recv.astype(jnp.int32).reshape(NW, NCH2, KC2)

    Wp = jnp.concatenate(
        [W_msg, A_s[:, :, None], A_r[:, :, None]], axis=2)

    h, hw, ab = _encode(world_pos, image_feature, W_enc1, b_enc1,
                        W_enc2, b_enc2, Wp[0])

    W1h = W_u1[:, :D, :]
    W1a = W_u1[:, D:, :]

    for i in range(S):
        ae_r = ae_T[i].reshape(NW, NCH, KC)
        ex_r, denp = _att_call(send_r, recv_r, ae_r, ab.reshape(-1))
        aggp = _msg_call(send_r2, recv_r2, ex_r, hw)
        Wp_next = Wp[i + 1] if i + 1 < S else Wp[0]
        h, hw, ab = _update(h, aggp, aggp,
                            denp[0, :N, None], denp[1, :N, None],
                            W1h[i], W1a[i], b_u1[i],
                            W_u2[i], b_u2[i], Wp_next)

    return _decode(h, W_d1, b_d1, W_d2, b_d2)


# speedup vs baseline: 22.7220x; 1.0858x over previous
import functools

import jax
import jax.numpy as jnp
from jax import lax
from jax.experimental import pallas as pl
from jax.experimental.pallas import tpu as pltpu
from jax.experimental.pallas import tpu_sc as plsc

N = 10000
D = 128
NBLK = 2000
EBLK = 16000

NC = 2
NS = 16
NW = NC * NS
E2 = 320000
EW = E2 // NW
KC = 80
NCH = EW // KC
KC2 = 80
NCH2 = EW // KC2
NROW = 640


def _full(shape):
    return pl.BlockSpec(shape, lambda i: (0,) * len(shape))


def _encode_body(pos_ref, img_ref, W1p_ref, W1i_ref, b1_ref, W2_ref, b2_ref,
                 Wp_ref, h_ref, hw_ref, ab_ref):
    t = jnp.maximum(pos_ref[...] @ W1p_ref[...] + img_ref[...] @ W1i_ref[...]
                    + b1_ref[...], 0.0)
    h = t @ W2_ref[...] + b2_ref[...]
    h_ref[...] = h
    p = h @ Wp_ref[...]
    hw_ref[...] = p[:, :D]
    ab_ref[...] = p[:, D:]


def _encode(pos, img, W1, b1, W2, b2, Wp):
    grid = (N // NBLK,)
    return pl.pallas_call(
        _encode_body,
        grid=grid,
        in_specs=[
            pl.BlockSpec((NBLK, 3), lambda i: (i, 0)),
            pl.BlockSpec((NBLK, D), lambda i: (i, 0)),
            _full((3, D)), _full((D, D)), _full((1, D)),
            _full((D, D)), _full((1, D)), _full((D, D + 2)),
        ],
        out_specs=[
            pl.BlockSpec((NBLK, D), lambda i: (i, 0)),
            pl.BlockSpec((NBLK, D), lambda i: (i, 0)),
            pl.BlockSpec((NBLK, 2), lambda i: (i, 0)),
        ],
        out_shape=[
            jax.ShapeDtypeStruct((N, D), jnp.float32),
            jax.ShapeDtypeStruct((N, D), jnp.float32),
            jax.ShapeDtypeStruct((N, 2), jnp.float32),
        ],
    )(pos, img, W1[:3], W1[3:], b1[None], W2, b2[None], Wp)


def _ae_body(ef_ref, W1_ref, b1_ref, B_ref, c_ref, out_ref):
    t = jnp.maximum(ef_ref[...] @ W1_ref[...] + b1_ref[...], 0.0)
    out_ref[...] = t @ B_ref[...] + c_ref[...]


def _ae_table(efeat, W_eenc1, b_eenc1, B_e, c_e, E2, S):
    grid = (E2 // EBLK,)
    return pl.pallas_call(
        _ae_body,
        grid=grid,
        in_specs=[
            pl.BlockSpec((EBLK, 4), lambda i: (i, 0)),
            _full((4, D)), _full((1, D)), _full((D, S)), _full((1, S)),
        ],
        out_specs=pl.BlockSpec((EBLK, S), lambda i: (i, 0)),
        out_shape=jax.ShapeDtypeStruct((E2, S), jnp.float32),
    )(efeat, W_eenc1, b_eenc1[None], B_e, c_e)


def _update_body(h_ref, a0_ref, a1_ref, d0_ref, d1_ref, W1h_ref, W1a_ref,
                 b1_ref, W2_ref, b2_ref, Wp_ref, hn_ref, hw_ref, ab_ref):
    den = d0_ref[...] + d1_ref[...] + 1e-9
    agg = (a0_ref[0] + a1_ref[0]) / den
    t = jnp.maximum(h_ref[...] @ W1h_ref[...] + agg @ W1a_ref[...]
                    + b1_ref[...], 0.0)
    hn = h_ref[...] + t @ W2_ref[...] + b2_ref[...]
    hn_ref[...] = hn
    p = hn @ Wp_ref[...]
    hw_ref[...] = p[:, :D]
    ab_ref[...] = p[:, D:]


def _update(h, a0, a1, d0, d1, W1h, W1a, b1, W2, b2, Wp):
    grid = (N // NBLK,)
    return pl.pallas_call(
        _update_body,
        grid=grid,
        in_specs=[
            pl.BlockSpec((NBLK, D), lambda i: (i, 0)),
            pl.BlockSpec((1, NBLK, D), lambda i: (0, i, 0)),
            pl.BlockSpec((1, NBLK, D), lambda i: (1, i, 0)),
            pl.BlockSpec((NBLK, 1), lambda i: (i, 0)),
            pl.BlockSpec((NBLK, 1), lambda i: (i, 0)),
            _full((D, D)), _full((D, D)), _full((1, D)),
            _full((D, D)), _full((1, D)), _full((D, D + 2)),
        ],
        out_specs=[
            pl.BlockSpec((NBLK, D), lambda i: (i, 0)),
            pl.BlockSpec((NBLK, D), lambda i: (i, 0)),
            pl.BlockSpec((NBLK, 2), lambda i: (i, 0)),
        ],
        out_shape=[
            jax.ShapeDtypeStruct((N, D), jnp.float32),
            jax.ShapeDtypeStruct((N, D), jnp.float32),
            jax.ShapeDtypeStruct((N, 2), jnp.float32),
        ],
    )(h, a0, a1, d0, d1, W1h, W1a, b1[None], W2, b2[None], Wp)


def _decode_body(h_ref, W1_ref, b1_ref, W2_ref, b2_ref, out_ref):
    t = jnp.maximum(h_ref[...] @ W1_ref[...] + b1_ref[...], 0.0)
    out_ref[...] = t @ W2_ref[...] + b2_ref[...]


def _decode(h, W1, b1, W2, b2):
    grid = (N // NBLK,)
    return pl.pallas_call(
        _decode_body,
        grid=grid,
        in_specs=[
            pl.BlockSpec((NBLK, D), lambda i: (i, 0)),
            _full((D, D)), _full((1, D)), _full((D, 3)), _full((1, 3)),
        ],
        out_specs=pl.BlockSpec((NBLK, 3), lambda i: (i, 0)),
        out_shape=jax.ShapeDtypeStruct((N, 3), jnp.float32),
    )(h, W1, b1[None], W2, b2[None])


_MESH = None


def _mesh():
    global _MESH
    if _MESH is None:
        _MESH = plsc.VectorSubcoreMesh(core_axis_name="c", subcore_axis_name="s")
    return _MESH


NP = NROW * 16


def _att_body(send_h, recv_h, ae_h, ab_h, ex_h, denp_h,
              send_v, recv_v, ae_v, ex_v, ab_v, denp_v, tmp_v, acc_v,
              parts_sh):
    c = lax.axis_index("c")
    s = lax.axis_index("s")
    wid = c * NS + s
    pltpu.sync_copy(send_h.at[wid], send_v)
    pltpu.sync_copy(recv_h.at[wid], recv_v)
    pltpu.sync_copy(ae_h.at[wid], ae_v)
    pltpu.sync_copy(ab_h, ab_v)

    zero16 = jnp.zeros((16,), jnp.float32)

    def zbody(k, _):
        denp_v[pl.ds(k * 16, 16)] = zero16
        return 0
    lax.fori_loop(0, NROW, zbody, 0)

    def body(j, _):
        for t in range(KC // 16):
            sl = pl.ds(t * 16, 16)
            s16 = send_v[j, sl]
            r16 = recv_v[j, sl]
            x = (plsc.load_gather(ab_v, [s16 * 2])
                 + plsc.load_gather(ab_v, [r16 * 2 + 1])
                 + ae_v[j, sl])
            l = jnp.where(x >= 0.0, x, 0.2 * x)
            e = jnp.exp(l)
            ex_v[pl.ds(j * KC + t * 16, 16)] = e
            plsc.addupdate_scatter(denp_v, [r16], e)
        return 0
    lax.fori_loop(0, NCH, body, 0)

    pltpu.sync_copy(ex_v, ex_h.at[pl.ds(wid * EW, EW)])
    pltpu.sync_copy(denp_v, parts_sh.at[s])
    plsc.subcore_barrier()
    seg = NP // NS

    def zacc(t, _):
        acc_v[pl.ds(t * 16, 16)] = zero16
        return 0
    lax.fori_loop(0, seg // 16, zacc, 0)
    for p in range(NS):
        pltpu.sync_copy(parts_sh.at[p, pl.ds(s * seg, seg)], tmp_v)

        def radd(t, _):
            sl = pl.ds(t * 16, 16)
            acc_v[sl] = acc_v[sl] + tmp_v[sl]
            return 0
        lax.fori_loop(0, seg // 16, radd, 0)
    pltpu.sync_copy(acc_v, denp_h.at[c, pl.ds(s * seg, seg)])


_ATT_K = None


def _att_call(send_r, recv_r, ae_r, ab_flat):
    global _ATT_K
    if _ATT_K is None:
        _ATT_K = _make_att()
    return _ATT_K(send_r, recv_r, ae_r, ab_flat)


def _make_att():
    return pl.kernel(
        _att_body,
        out_type=[
            jax.ShapeDtypeStruct((NW * EW,), jnp.float32),
            jax.ShapeDtypeStruct((NC, NP), jnp.float32),
        ],
        mesh=_mesh(),
        compiler_params=pltpu.CompilerParams(needs_layout_passes=False),
        scratch_types=[
            pltpu.VMEM((NCH, KC), jnp.int32),
            pltpu.VMEM((NCH, KC), jnp.int32),
            pltpu.VMEM((NCH, KC), jnp.float32),
            pltpu.VMEM((EW,), jnp.float32),
            pltpu.VMEM((2 * N,), jnp.float32),
            pltpu.VMEM((NP,), jnp.float32),
            pltpu.VMEM((NP // NS,), jnp.float32),
            pltpu.VMEM((NP // NS,), jnp.float32),
            pltpu.VMEM_SHARED((NS, NP), jnp.float32),
        ],
    )


def _msg_body(send_h, recv_h, ex_h, hw_h, aggp_h,
              recv_v, sb0, sb1, sb2, eb0, eb1, eb2, r0_v, r1_v, r2_v,
              i0, i1, i2, g0, g1, g2, s0, s1, s2, agg_sh):
    c = lax.axis_index("c")
    s = lax.axis_index("s")
    wid = c * NS + s
    pltpu.sync_copy(recv_h.at[wid], recv_v)

    sendb = (sb0, sb1, sb2)
    exb = (eb0, eb1, eb2)
    rows = (r0_v, r1_v, r2_v)
    isem = (i0, i1, i2)
    gsem = (g0, g1, g2)
    ssem = (s0, s1, s2)

    zero16 = jnp.zeros((16,), jnp.float32)

    def zbody(k, _):
        r0_v[lax.shift_right_logical(k, 3),
             pl.ds(lax.bitwise_and(k, 7) * 16, 16)] = zero16
        return 0
    lax.fori_loop(0, KC2 * 8, zbody, 0)
    for q in range(640 // KC2):
        pltpu.sync_copy(r0_v, agg_sh.at[pl.ds(s * 640 + q * KC2, KC2)])
    plsc.subcore_barrier()

    def fire_ise(j, b):
        pltpu.async_copy(send_h.at[wid, j], sendb[b], isem[b])
        pltpu.async_copy(ex_h.at[pl.ds(wid * EW + j * KC2, KC2)], exb[b],
                         isem[b])

    def wait_ise(b):
        pltpu.make_async_copy(send_h.at[wid, 0], sendb[b], isem[b]).wait()
        pltpu.make_async_copy(ex_h.at[pl.ds(0, KC2)], exb[b], isem[b]).wait()

    def fire_g(b):
        pltpu.async_copy(hw_h.at[sendb[b]], rows[b], gsem[b])

    def wait_g(b):
        pltpu.make_async_copy(hw_h.at[sendb[b]], rows[b], gsem[b]).wait()

    def fire_s(j, b):
        pltpu.async_copy(rows[b], agg_sh.at[recv_v.at[j]], ssem[b], add=True)

    def wait_s(b):
        pltpu.make_async_copy(rows[b], agg_sh.at[recv_v.at[0]],
                              ssem[b]).wait()

    fire_ise(0, 0)
    fire_ise(1, 1)
    wait_ise(0)
    fire_g(0)

    def qbody(q, _):
        for b in range(3):
            j = 3 * q + b
            b1 = (b + 1) % 3
            b2 = (b + 2) % 3

            @pl.when(j + 1 < NCH2)
            def _():
                wait_ise(b1)

                @pl.when(j >= 2)
                def _():
                    wait_s(b1)
                fire_g(b1)

            @pl.when(j + 2 < NCH2)
            def _():
                fire_ise(j + 2, b2)

            @pl.when(j < NCH2)
            def _():
                wait_g(b)

                def rbody(k, _):
                    a16 = plsc.load_gather(exb[b],
                                           [jnp.full((16,), k, jnp.int32)])
                    for u in range(8):
                        su = pl.ds(u * 16, 16)
                        rows[b][k, su] = rows[b][k, su] * a16
                    return 0
                lax.fori_loop(0, KC2, rbody, 0)
                fire_s(j, b)
        return 0
    lax.fori_loop(0, (NCH2 + 2) // 3, qbody, 0)
    wait_s((NCH2 - 3) % 3)
    wait_s((NCH2 - 2) % 3)
    wait_s((NCH2 - 1) % 3)
    plsc.subcore_barrier()

    for q in range(5):
        pltpu.sync_copy(agg_sh.at[pl.ds(s * 640 + q * 128, 128)],
                        aggp_h.at[c, pl.ds(s * 640 + q * 128, 128)])


_MSG_K = None


def _msg_call(send_r, recv_r, ex_r, hw):
    global _MSG_K
    if _MSG_K is None:
        _MSG_K = _make_msg()
    return _MSG_K(send_r, recv_r, ex_r, hw)


def _make_msg():
    return pl.kernel(
        _msg_body,
        out_type=jax.ShapeDtypeStruct((NC, NP, D), jnp.float32),
        mesh=_mesh(),
        compiler_params=pltpu.CompilerParams(needs_layout_passes=False),
        scratch_types=[
            pltpu.VMEM((NCH2, KC2), jnp.int32),
            pltpu.VMEM((KC2,), jnp.int32),
            pltpu.VMEM((KC2,), jnp.int32),
            pltpu.VMEM((KC2,), jnp.int32),
            pltpu.VMEM((KC2,), jnp.float32),
            pltpu.VMEM((KC2,), jnp.float32),
            pltpu.VMEM((KC2,), jnp.float32),
            pltpu.VMEM((KC2, D), jnp.float32),
            pltpu.VMEM((KC2, D), jnp.float32),
            pltpu.VMEM((KC2, D), jnp.float32),
            pltpu.SemaphoreType.DMA,
            pltpu.SemaphoreType.DMA,
            pltpu.SemaphoreType.DMA,
            pltpu.SemaphoreType.DMA,
            pltpu.SemaphoreType.DMA,
            pltpu.SemaphoreType.DMA,
            pltpu.SemaphoreType.DMA,
            pltpu.SemaphoreType.DMA,
            pltpu.SemaphoreType.DMA,
            pltpu.VMEM_SHARED((NP, D), jnp.float32),
        ],
    )


def kernel(world_pos, image_feature, edge_index, is_training,
           read_intermediate, vis_att,
           W_enc1, b_enc1, W_enc2, b_enc2, W_eenc1, b_eenc1, W_eenc2, b_eenc2,
           A_s, A_r, A_e, W_msg, W_u1, b_u1, W_u2, b_u2, W_d1, b_d1, W_d2, b_d2):
    S = A_s.shape[0]
    s0 = edge_index[0]
    r0 = edge_index[1]
    send = jnp.concatenate([s0, r0], 0)
    recv = jnp.concatenate([r0, s0], 0)
    E2 = send.shape[0]

    rel = world_pos[send] - world_pos[recv]
    nrm = jnp.linalg.norm(rel, axis=-1, keepdims=True)
    efeat = jnp.concatenate([rel, nrm], -1)

    B_e = W_eenc2 @ A_e.T
    c_e = (b_eenc2 @ A_e.T)[None]
    ae_all = _ae_table(efeat, W_eenc1, b_eenc1, B_e, c_e, E2, S)
    ae_T = ae_all.T

    send_r = send.astype(jnp.int32).reshape(NW, NCH, KC)
    recv_r = recv.astype(jnp.int32).reshape(NW, NCH, KC)
    send_r2 = send.astype(jnp.int32).reshape(NW, NCH2, KC2)
    recv_r2 = recv.astype(jnp.int32).reshape(NW, NCH2, KC2)

    Wp = jnp.concatenate(
        [W_msg, A_s[:, :, None], A_r[:, :, None]], axis=2)

    h, hw, ab = _encode(world_pos, image_feature, W_enc1, b_enc1,
                        W_enc2, b_enc2, Wp[0])

    W1h = W_u1[:, :D, :]
    W1a = W_u1[:, D:, :]

    for i in range(S):
        ae_r = ae_T[i].reshape(NW, NCH, KC)
        ex_r, denp = _att_call(send_r, recv_r, ae_r, ab.reshape(-1))
        aggp = _msg_call(send_r2, recv_r2, ex_r, hw)
        Wp_next = Wp[i + 1] if i + 1 < S else Wp[0]
        h, hw, ab = _update(h, aggp, aggp,
                            denp[0, :N, None], denp[1, :N, None],
                            W1h[i], W1a[i], b_u1[i],
                            W_u2[i], b_u2[i], Wp_next)

    return _decode(h, W_d1, b_d1, W_d2, b_d2)


# speedup vs baseline: 22.8021x; 1.0035x over previous
import functools

import jax
import jax.numpy as jnp
from jax import lax
from jax.experimental import pallas as pl
from jax.experimental.pallas import tpu as pltpu
from jax.experimental.pallas import tpu_sc as plsc

N = 10000
D = 128
NBLK = 2000
EBLK = 16000

NC = 2
NS = 16
NW = NC * NS
E2 = 320000
EW = E2 // NW
KC = 80
NCH = EW // KC
KC2 = 80
NCH2 = EW // KC2
NROW = 640


def _full(shape):
    return pl.BlockSpec(shape, lambda i: (0,) * len(shape))


def _encode_body(pos_ref, img_ref, W1p_ref, W1i_ref, b1_ref, W2_ref, b2_ref,
                 Wp_ref, h_ref, hw_ref, ab_ref):
    t = jnp.maximum(pos_ref[...] @ W1p_ref[...] + img_ref[...] @ W1i_ref[...]
                    + b1_ref[...], 0.0)
    h = t @ W2_ref[...] + b2_ref[...]
    h_ref[...] = h
    p = h @ Wp_ref[...]
    hw_ref[...] = p[:, :D]
    ab_ref[...] = p[:, D:]


def _encode(pos, img, W1, b1, W2, b2, Wp):
    grid = (N // NBLK,)
    return pl.pallas_call(
        _encode_body,
        grid=grid,
        in_specs=[
            pl.BlockSpec((NBLK, 3), lambda i: (i, 0)),
            pl.BlockSpec((NBLK, D), lambda i: (i, 0)),
            _full((3, D)), _full((D, D)), _full((1, D)),
            _full((D, D)), _full((1, D)), _full((D, D + 2)),
        ],
        out_specs=[
            pl.BlockSpec((NBLK, D), lambda i: (i, 0)),
            pl.BlockSpec((NBLK, D), lambda i: (i, 0)),
            pl.BlockSpec((NBLK, 2), lambda i: (i, 0)),
        ],
        out_shape=[
            jax.ShapeDtypeStruct((N, D), jnp.float32),
            jax.ShapeDtypeStruct((N, D), jnp.float32),
            jax.ShapeDtypeStruct((N, 2), jnp.float32),
        ],
    )(pos, img, W1[:3], W1[3:], b1[None], W2, b2[None], Wp)


def _ae_body(ef_ref, W1_ref, b1_ref, B_ref, c_ref, out_ref):
    t = jnp.maximum(ef_ref[...] @ W1_ref[...] + b1_ref[...], 0.0)
    out_ref[...] = t @ B_ref[...] + c_ref[...]


def _ae_table(efeat, W_eenc1, b_eenc1, B_e, c_e, E2, S):
    grid = (E2 // EBLK,)
    return pl.pallas_call(
        _ae_body,
        grid=grid,
        in_specs=[
            pl.BlockSpec((EBLK, 4), lambda i: (i, 0)),
            _full((4, D)), _full((1, D)), _full((D, S)), _full((1, S)),
        ],
        out_specs=pl.BlockSpec((EBLK, S), lambda i: (i, 0)),
        out_shape=jax.ShapeDtypeStruct((E2, S), jnp.float32),
    )(efeat, W_eenc1, b_eenc1[None], B_e, c_e)


def _update_body(h_ref, a0_ref, a1_ref, d_ref, W1h_ref, W1a_ref,
                 b1_ref, W2_ref, b2_ref, Wp_ref, hn_ref, hw_ref, ab_ref):
    agg = (a0_ref[0] + a1_ref[0]) / d_ref[...]
    t = jnp.maximum(h_ref[...] @ W1h_ref[...] + agg @ W1a_ref[...]
                    + b1_ref[...], 0.0)
    hn = h_ref[...] + t @ W2_ref[...] + b2_ref[...]
    hn_ref[...] = hn
    p = hn @ Wp_ref[...]
    hw_ref[...] = p[:, :D]
    ab_ref[...] = p[:, D:]


def _update(h, a0, a1, d, W1h, W1a, b1, W2, b2, Wp):
    grid = (N // NBLK,)
    return pl.pallas_call(
        _update_body,
        grid=grid,
        in_specs=[
            pl.BlockSpec((NBLK, D), lambda i: (i, 0)),
            pl.BlockSpec((1, NBLK, D), lambda i: (0, i, 0)),
            pl.BlockSpec((1, NBLK, D), lambda i: (1, i, 0)),
            pl.BlockSpec((NBLK, 1), lambda i: (i, 0)),
            _full((D, D)), _full((D, D)), _full((1, D)),
            _full((D, D)), _full((1, D)), _full((D, D + 2)),
        ],
        out_specs=[
            pl.BlockSpec((NBLK, D), lambda i: (i, 0)),
            pl.BlockSpec((NBLK, D), lambda i: (i, 0)),
            pl.BlockSpec((NBLK, 2), lambda i: (i, 0)),
        ],
        out_shape=[
            jax.ShapeDtypeStruct((N, D), jnp.float32),
            jax.ShapeDtypeStruct((N, D), jnp.float32),
            jax.ShapeDtypeStruct((N, 2), jnp.float32),
        ],
    )(h, a0, a1, d, W1h, W1a, b1[None], W2, b2[None], Wp)


def _decode_body(h_ref, W1_ref, b1_ref, W2_ref, b2_ref, out_ref):
    t = jnp.maximum(h_ref[...] @ W1_ref[...] + b1_ref[...], 0.0)
    out_ref[...] = t @ W2_ref[...] + b2_ref[...]


def _decode(h, W1, b1, W2, b2):
    grid = (N // NBLK,)
    return pl.pallas_call(
        _decode_body,
        grid=grid,
        in_specs=[
            pl.BlockSpec((NBLK, D), lambda i: (i, 0)),
            _full((D, D)), _full((1, D)), _full((D, 3)), _full((1, 3)),
        ],
        out_specs=pl.BlockSpec((NBLK, 3), lambda i: (i, 0)),
        out_shape=jax.ShapeDtypeStruct((N, 3), jnp.float32),
    )(h, W1, b1[None], W2, b2[None])


_MESH = None


def _mesh():
    global _MESH
    if _MESH is None:
        _MESH = plsc.VectorSubcoreMesh(core_axis_name="c", subcore_axis_name="s")
    return _MESH


NP = NROW * 16


def _att_body(send_h, recv_h, ae_h, ab_h, ex_h, denp_h,
              send_v, recv_v, ae_v, ex_v, ab_v, denp_v, tmp_v, tmp_v2, acc_v,
              rsem, parts_sh):
    c = lax.axis_index("c")
    s = lax.axis_index("s")
    wid = c * NS + s
    pltpu.sync_copy(send_h.at[wid], send_v)
    pltpu.sync_copy(recv_h.at[wid], recv_v)
    pltpu.sync_copy(ae_h.at[wid], ae_v)
    pltpu.sync_copy(ab_h, ab_v)

    zero16 = jnp.zeros((16,), jnp.float32)

    def zbody(k, _):
        denp_v[pl.ds(k * 16, 16)] = zero16
        return 0
    lax.fori_loop(0, NROW, zbody, 0)

    def body(j, _):
        for t in range(KC // 16):
            sl = pl.ds(t * 16, 16)
            s16 = send_v[j, sl]
            r16 = recv_v[j, sl]
            x = (plsc.load_gather(ab_v, [s16 * 2])
                 + plsc.load_gather(ab_v, [r16 * 2 + 1])
                 + ae_v[j, sl])
            l = jnp.where(x >= 0.0, x, 0.2 * x)
            e = jnp.exp(l)
            ex_v[pl.ds(j * KC + t * 16, 16)] = e
            plsc.addupdate_scatter(denp_v, [r16], e)
        return 0
    lax.fori_loop(0, NCH, body, 0)

    pltpu.sync_copy(ex_v, ex_h.at[pl.ds(wid * EW, EW)])
    pltpu.sync_copy(denp_v, parts_sh.at[s])
    plsc.subcore_barrier()
    seg = NP // NS

    def zacc(t, _):
        acc_v[pl.ds(t * 16, 16)] = zero16
        return 0
    lax.fori_loop(0, seg // 16, zacc, 0)
    tmp2 = (tmp_v, tmp_v2)
    pltpu.async_copy(parts_sh.at[0, pl.ds(s * seg, seg)], tmp_v, rsem)
    for p in range(NS):
        pltpu.make_async_copy(parts_sh.at[p, pl.ds(s * seg, seg)],
                              tmp2[p % 2], rsem).wait()
        if p + 1 < NS:
            pltpu.async_copy(parts_sh.at[p + 1, pl.ds(s * seg, seg)],
                             tmp2[(p + 1) % 2], rsem)

        def radd(t, _):
            sl = pl.ds(t * 16, 16)
            acc_v[sl] = acc_v[sl] + tmp2[p % 2][sl]
            return 0
        lax.fori_loop(0, seg // 16, radd, 0)
    pltpu.sync_copy(acc_v, denp_h.at[c, pl.ds(s * seg, seg)])


_ATT_K = None


def _att_call(send_r, recv_r, ae_r, ab_flat):
    global _ATT_K
    if _ATT_K is None:
        _ATT_K = _make_att()
    return _ATT_K(send_r, recv_r, ae_r, ab_flat)


def _make_att():
    return pl.kernel(
        _att_body,
        out_type=[
            jax.ShapeDtypeStruct((NW * EW,), jnp.float32),
            jax.ShapeDtypeStruct((NC, NP), jnp.float32),
        ],
        mesh=_mesh(),
        compiler_params=pltpu.CompilerParams(needs_layout_passes=False),
        scratch_types=[
            pltpu.VMEM((NCH, KC), jnp.int32),
            pltpu.VMEM((NCH, KC), jnp.int32),
            pltpu.VMEM((NCH, KC), jnp.float32),
            pltpu.VMEM((EW,), jnp.float32),
            pltpu.VMEM((2 * N,), jnp.float32),
            pltpu.VMEM((NP,), jnp.float32),
            pltpu.VMEM((NP // NS,), jnp.float32),
            pltpu.VMEM((NP // NS,), jnp.float32),
            pltpu.VMEM((NP // NS,), jnp.float32),
            pltpu.SemaphoreType.DMA,
            pltpu.VMEM_SHARED((NS, NP), jnp.float32),
        ],
    )


def _msg_body(send_h, recv_h, ex_h, hw_h, aggp_h,
              recv_v, sb0, sb1, sb2, eb0, eb1, eb2, r0_v, r1_v, r2_v,
              i0, i1, i2, g0, g1, g2, s0, s1, s2, agg_sh):
    c = lax.axis_index("c")
    s = lax.axis_index("s")
    wid = c * NS + s
    pltpu.sync_copy(recv_h.at[wid], recv_v)

    sendb = (sb0, sb1, sb2)
    exb = (eb0, eb1, eb2)
    rows = (r0_v, r1_v, r2_v)
    isem = (i0, i1, i2)
    gsem = (g0, g1, g2)
    ssem = (s0, s1, s2)

    zero16 = jnp.zeros((16,), jnp.float32)

    def zbody(k, _):
        r0_v[lax.shift_right_logical(k, 3),
             pl.ds(lax.bitwise_and(k, 7) * 16, 16)] = zero16
        return 0
    lax.fori_loop(0, KC2 * 8, zbody, 0)
    for q in range(640 // KC2):
        pltpu.sync_copy(r0_v, agg_sh.at[pl.ds(s * 640 + q * KC2, KC2)])
    plsc.subcore_barrier()

    def fire_ise(j, b):
        pltpu.async_copy(send_h.at[wid, j], sendb[b], isem[b])
        pltpu.async_copy(ex_h.at[pl.ds(wid * EW + j * KC2, KC2)], exb[b],
                         isem[b])

    def wait_ise(b):
        pltpu.make_async_copy(send_h.at[wid, 0], sendb[b], isem[b]).wait()
        pltpu.make_async_copy(ex_h.at[pl.ds(0, KC2)], exb[b], isem[b]).wait()

    def fire_g(b):
        pltpu.async_copy(hw_h.at[sendb[b]], rows[b], gsem[b])

    def wait_g(b):
        pltpu.make_async_copy(hw_h.at[sendb[b]], rows[b], gsem[b]).wait()

    def fire_s(j, b):
        pltpu.async_copy(rows[b], agg_sh.at[recv_v.at[j]], ssem[b], add=True)

    def wait_s(b):
        pltpu.make_async_copy(rows[b], agg_sh.at[recv_v.at[0]],
                              ssem[b]).wait()

    fire_ise(0, 0)
    fire_ise(1, 1)
    wait_ise(0)
    fire_g(0)

    def qbody(q, _):
        for b in range(3):
            j = 3 * q + b
            b1 = (b + 1) % 3
            b2 = (b + 2) % 3

            @pl.when(j + 1 < NCH2)
            def _():
                wait_ise(b1)

                @pl.when(j >= 2)
                def _():
                    wait_s(b1)
                fire_g(b1)

            @pl.when(j + 2 < NCH2)
            def _():
                fire_ise(j + 2, b2)

            @pl.when(j < NCH2)
            def _():
                wait_g(b)

                def rbody(k, _):
                    a16 = plsc.load_gather(exb[b],
                                           [jnp.full((16,), k, jnp.int32)])
                    for u in range(8):
                        su = pl.ds(u * 16, 16)
                        rows[b][k, su] = rows[b][k, su] * a16
                    return 0
                lax.fori_loop(0, KC2, rbody, 0)
                fire_s(j, b)
        return 0
    lax.fori_loop(0, (NCH2 + 2) // 3, qbody, 0)
    wait_s((NCH2 - 3) % 3)
    wait_s((NCH2 - 2) % 3)
    wait_s((NCH2 - 1) % 3)
    plsc.subcore_barrier()

    for q in range(5):
        pltpu.sync_copy(agg_sh.at[pl.ds(s * 640 + q * 128, 128)],
                        aggp_h.at[c, pl.ds(s * 640 + q * 128, 128)])


_MSG_K = None


def _msg_call(send_r, recv_r, ex_r, hw):
    global _MSG_K
    if _MSG_K is None:
        _MSG_K = _make_msg()
    return _MSG_K(send_r, recv_r, ex_r, hw)


def _make_msg():
    return pl.kernel(
        _msg_body,
        out_type=jax.ShapeDtypeStruct((NC, NP, D), jnp.float32),
        mesh=_mesh(),
        compiler_params=pltpu.CompilerParams(needs_layout_passes=False),
        scratch_types=[
            pltpu.VMEM((NCH2, KC2), jnp.int32),
            pltpu.VMEM((KC2,), jnp.int32),
            pltpu.VMEM((KC2,), jnp.int32),
            pltpu.VMEM((KC2,), jnp.int32),
            pltpu.VMEM((KC2,), jnp.float32),
            pltpu.VMEM((KC2,), jnp.float32),
            pltpu.VMEM((KC2,), jnp.float32),
            pltpu.VMEM((KC2, D), jnp.float32),
            pltpu.VMEM((KC2, D), jnp.float32),
            pltpu.VMEM((KC2, D), jnp.float32),
            pltpu.SemaphoreType.DMA,
            pltpu.SemaphoreType.DMA,
            pltpu.SemaphoreType.DMA,
            pltpu.SemaphoreType.DMA,
            pltpu.SemaphoreType.DMA,
            pltpu.SemaphoreType.DMA,
            pltpu.SemaphoreType.DMA,
            pltpu.SemaphoreType.DMA,
            pltpu.SemaphoreType.DMA,
            pltpu.VMEM_SHARED((NP, D), jnp.float32),
        ],
    )


def kernel(world_pos, image_feature, edge_index, is_training,
           read_intermediate, vis_att,
           W_enc1, b_enc1, W_enc2, b_enc2, W_eenc1, b_eenc1, W_eenc2, b_eenc2,
           A_s, A_r, A_e, W_msg, W_u1, b_u1, W_u2, b_u2, W_d1, b_d1, W_d2, b_d2):
    S = A_s.shape[0]
    s0 = edge_index[0]
    r0 = edge_index[1]
    send = jnp.concatenate([s0, r0], 0)
    recv = jnp.concatenate([r0, s0], 0)
    E2 = send.shape[0]

    rel = world_pos[send] - world_pos[recv]
    nrm = jnp.linalg.norm(rel, axis=-1, keepdims=True)
    efeat = jnp.concatenate([rel, nrm], -1)

    B_e = W_eenc2 @ A_e.T
    c_e = (b_eenc2 @ A_e.T)[None]
    ae_all = _ae_table(efeat, W_eenc1, b_eenc1, B_e, c_e, E2, S)
    ae_T = ae_all.T

    send_r = send.astype(jnp.int32).reshape(NW, NCH, KC)
    recv_r = recv.astype(jnp.int32).reshape(NW, NCH, KC)
    send_r2 = send.astype(jnp.int32).reshape(NW, NCH2, KC2)
    recv_r2 = recv.astype(jnp.int32).reshape(NW, NCH2, KC2)

    Wp = jnp.concatenate(
        [W_msg, A_s[:, :, None], A_r[:, :, None]], axis=2)

    h, hw, ab = _encode(world_pos, image_feature, W_enc1, b_enc1,
                        W_enc2, b_enc2, Wp[0])

    W1h = W_u1[:, :D, :]
    W1a = W_u1[:, D:, :]

    for i in range(S):
        ae_r = ae_T[i].reshape(NW, NCH, KC)
        ex_r, denp = _att_call(send_r, recv_r, ae_r, ab.reshape(-1))
        aggp = _msg_call(send_r2, recv_r2, ex_r, hw)
        Wp_next = Wp[i + 1] if i + 1 < S else Wp[0]
        den = (denp[0] + denp[1] + 1e-9)[:N, None]
        h, hw, ab = _update(h, aggp, aggp, den,
                            W1h[i], W1a[i], b_u1[i],
                            W_u2[i], b_u2[i], Wp_next)

    return _decode(h, W_d1, b_d1, W_d2, b_d2)
